# baseline (speedup 1.0000x reference)
import functools

import jax
import jax.numpy as jnp
from jax import lax
from jax.experimental import pallas as pl
from jax.experimental.pallas import tpu as pltpu
from jax.experimental.pallas import tpu_sc as plsc

N_ATOM = 10000
ATOM_DIM = 128
MSG = 64
E = 320000
N_FACTOR = 160000

NC, NS = 2, 16
NW = NC * NS
CH = 80
EPW = E // NW
CPW = EPW // CH
KD = 5
NG = CPW // KD
N_ATOM_PAD = 10240
ROWS_PER_TILE = N_ATOM_PAD // NS

_f32 = jnp.float32


def _atom_table_body(x_ref, wb, wa, wt, bb, ba, bt, hb, ha, ht):
    xv = x_ref[...]
    hb[...] = jnp.dot(xv, wb[...], preferred_element_type=_f32) + bb[...]
    ha[...] = jnp.dot(xv, wa[...], preferred_element_type=_f32) + ba[...]
    ht[...] = jnp.dot(xv, wt[...], preferred_element_type=_f32) + bt[...]


def _atom_tables(x, ws, bs):
    blk = 1000
    grid = (N_ATOM // blk,)
    wspec = pl.BlockSpec((ATOM_DIM, MSG), lambda i: (0, 0))
    bspec = pl.BlockSpec((1, MSG), lambda i: (0, 0))
    hspec = pl.BlockSpec((blk, MSG), lambda i: (i, 0))
    return pl.pallas_call(
        _atom_table_body,
        grid=grid,
        in_specs=[pl.BlockSpec((blk, ATOM_DIM), lambda i: (i, 0))] + [wspec] * 3 + [bspec] * 3,
        out_specs=[hspec] * 3,
        out_shape=[jax.ShapeDtypeStruct((N_ATOM, MSG), _f32)] * 3,
    )(x, *ws, *bs)


def _sc_gather_body(hb, ha, ht, db, da, dt, sb, sa, st, fb, fa, ft,
                    hgb, hga, hgt, tgb, tga, tgt,
                    dstbuf, srcbuf, rowbuf, tfull, rsem, ssem, wsem):
    cid = lax.axis_index("c")
    sid = lax.axis_index("s")
    wid = cid * NS + sid
    for h_hbm, d3, s3, fx_hbm, hg_hbm, tg3 in (
        (hb, db, sb, fb, hgb, tgb),
        (ha, da, sa, fa, hga, tga),
        (ht, dt, st, ft, hgt, tgt),
    ):
        pltpu.sync_copy(d3.at[wid], dstbuf)
        pltpu.sync_copy(s3.at[wid], srcbuf)

        def group(g, _, h_hbm=h_hbm, fx_hbm=fx_hbm, hg_hbm=hg_hbm):
            descs = []
            for b in range(KD):
                i = g * KD + b
                descs.append(pltpu.async_copy(h_hbm.at[dstbuf.at[i]], rowbuf.at[b], rsem))
                descs.append(pltpu.async_copy(fx_hbm.at[srcbuf.at[i]], tfull.at[i], ssem))
            for d in descs:
                d.wait()
            descs = []
            for b in range(KD):
                e0 = (wid * CPW + g * KD + b) * CH
                descs.append(pltpu.async_copy(rowbuf.at[b], hg_hbm.at[pl.ds(e0, CH)], wsem))
            for d in descs:
                d.wait()
            return 0

        lax.fori_loop(0, NG, group, 0)
        pltpu.sync_copy(tfull, tg3.at[wid])


def _sc_gather(hs, d3s, s3s, fxs):
    mesh = plsc.VectorSubcoreMesh(core_axis_name="c", subcore_axis_name="s")
    out_type = (
        [jax.ShapeDtypeStruct((E, MSG), _f32)] * 3
        + [jax.ShapeDtypeStruct((NW, CPW, CH), _f32)] * 3
    )
    f = pl.kernel(
        _sc_gather_body,
        out_type=out_type,
        mesh=mesh,
        compiler_params=pltpu.CompilerParams(use_tc_tiling_on_sc=False),
        scratch_types=[
            pltpu.VMEM((CPW, CH), jnp.int32),
            pltpu.VMEM((CPW, CH), jnp.int32),
            pltpu.VMEM((KD, CH, MSG), _f32),
            pltpu.VMEM((CPW, CH), _f32),
            pltpu.SemaphoreType.DMA,
            pltpu.SemaphoreType.DMA,
            pltpu.SemaphoreType.DMA,
        ],
    )
    return f(*hs, *d3s, *s3s, *fxs)


PK = 4
PW = PK * MSG


def _edge_mlp_body(hg4_ref, t2_ref, v4_ref, w2d_ref, b2d_ref, w3d_ref, b3d_ref, m4_ref):
    hp = hg4_ref[...]
    hb_rows = hp.shape[0]
    t2 = t2_ref[0]
    rowb = jnp.broadcast_to(t2[:, None, :], (t2.shape[0], 128 // PK, 128))
    rowb = rowb.reshape(hb_rows, 128)
    km = jax.lax.broadcasted_iota(jnp.int32, (hb_rows, 128), 0) % (128 // PK)
    c = jax.lax.broadcasted_iota(jnp.int32, (hb_rows, 128), 1)
    tq = [jnp.sum(jnp.where(PK * km + j == c, rowb, 0.0), axis=1, keepdims=True)
          for j in range(PK)]
    c4 = jax.lax.broadcasted_iota(jnp.int32, (hb_rows, PW), 1) // MSG
    t_all = tq[PK - 1]
    for j in range(PK - 2, -1, -1):
        t_all = jnp.where(c4 == j, tq[j], t_all)
    a1 = jnp.maximum(hp + t_all * v4_ref[...], 0.0)
    a2 = jnp.maximum(jnp.dot(a1, w2d_ref[...], preferred_element_type=_f32) + b2d_ref[...], 0.0)
    m4_ref[...] = jnp.dot(a2, w3d_ref[...], preferred_element_type=_f32) + b3d_ref[...]


def _edge_mlp(hg4, t2, v4, w2d, b2d, w3d, b3d):
    blk = 3200
    grid = (E // blk,)
    wspec = pl.BlockSpec((PW, PW), lambda i: (0, 0))
    bspec = pl.BlockSpec((1, PW), lambda i: (0, 0))
    return pl.pallas_call(
        _edge_mlp_body,
        grid=grid,
        in_specs=[
            pl.BlockSpec((blk // PK, PW), lambda i: (i, 0)),
            pl.BlockSpec((1, blk // 128, 128), lambda i: (i, 0, 0)),
            bspec, wspec, bspec, wspec, bspec,
        ],
        out_specs=pl.BlockSpec((blk // PK, PW), lambda i: (i, 0)),
        out_shape=jax.ShapeDtypeStruct((E // PK, PW), _f32),
    )(hg4, t2, v4, w2d, b2d, w3d, b3d)


def _sc_scatter_body(mb, ma, mt, db, da, dt, zrows,
                     pb, pa, pt,
                     dstbuf, rowbuf, acc, lsem, asem):
    cid = lax.axis_index("c")
    sid = lax.axis_index("s")
    wid = cid * NS + sid
    arow = sid * ROWS_PER_TILE
    for m_hbm, d3, p_hbm in ((mb, db, pb), (ma, da, pa), (mt, dt, pt)):
        pltpu.sync_copy(d3.at[wid], dstbuf)
        pltpu.sync_copy(zrows, acc.at[pl.ds(arow, ROWS_PER_TILE)])
        plsc.subcore_barrier()

        def group(g, _, m_hbm=m_hbm):
            descs = []
            for b in range(KD):
                e0 = (wid * CPW + g * KD + b) * CH
                descs.append(pltpu.async_copy(m_hbm.at[pl.ds(e0, CH)], rowbuf.at[b], lsem))
            for d in descs:
                d.wait()
            descs = []
            for b in range(KD):
                i = g * KD + b
                descs.append(pltpu.async_copy(rowbuf.at[b], acc.at[dstbuf.at[i]], asem, add=True))
            for d in descs:
                d.wait()
            return 0

        lax.fori_loop(0, NG, group, 0)
        plsc.subcore_barrier()
        pltpu.sync_copy(acc.at[pl.ds(arow, ROWS_PER_TILE)],
                        p_hbm.at[pl.ds(cid * N_ATOM_PAD + arow, ROWS_PER_TILE)])


def _sc_scatter(ms, d3s, zrows):
    mesh = plsc.VectorSubcoreMesh(core_axis_name="c", subcore_axis_name="s")
    out_type = [jax.ShapeDtypeStruct((NC * N_ATOM_PAD, MSG), _f32)] * 3
    f = pl.kernel(
        _sc_scatter_body,
        out_type=out_type,
        mesh=mesh,
        compiler_params=pltpu.CompilerParams(use_tc_tiling_on_sc=False),
        scratch_types=[
            pltpu.VMEM((CPW, CH), jnp.int32),
            pltpu.VMEM((KD, CH, MSG), _f32),
            pltpu.VMEM_SHARED((N_ATOM_PAD, MSG), _f32),
            pltpu.SemaphoreType.DMA,
            pltpu.SemaphoreType.DMA,
        ],
    )
    return f(*ms, *d3s, zrows)


def _combine_body(pb, pa, pt, w1, b1, w2, b2, w3, b3, out):
    ab = pb[0] + pb[1]
    aa = pa[0] + pa[1]
    at = pt[0] + pt[1]
    y = (jnp.dot(ab, w1[0:MSG, :], preferred_element_type=_f32)
         + jnp.dot(aa, w1[MSG:2 * MSG, :], preferred_element_type=_f32)
         + jnp.dot(at, w1[2 * MSG:3 * MSG, :], preferred_element_type=_f32)
         + b1[...])
    y = jnp.maximum(y, 0.0)
    y = jnp.maximum(jnp.dot(y, w2[...], preferred_element_type=_f32) + b2[...], 0.0)
    out[...] = jnp.dot(y, w3[...], preferred_element_type=_f32) + b3[...]


def _combine(ps, cw):
    w1, b1, w2, b2, w3, b3 = cw
    blk = 1000
    grid = (N_ATOM // blk,)
    pspec = pl.BlockSpec((NC, blk, MSG), lambda i: (0, i, 0))
    return pl.pallas_call(
        _combine_body,
        grid=grid,
        in_specs=[pspec, pspec, pspec,
                  pl.BlockSpec((3 * MSG, MSG), lambda i: (0, 0)),
                  pl.BlockSpec((1, MSG), lambda i: (0, 0)),
                  pl.BlockSpec((MSG, MSG), lambda i: (0, 0)),
                  pl.BlockSpec((1, MSG), lambda i: (0, 0)),
                  pl.BlockSpec((MSG, ATOM_DIM), lambda i: (0, 0)),
                  pl.BlockSpec((1, ATOM_DIM), lambda i: (0, 0))],
        out_specs=pl.BlockSpec((blk, ATOM_DIM), lambda i: (i, 0)),
        out_shape=jax.ShapeDtypeStruct((N_ATOM, ATOM_DIM), _f32),
    )(ps[0].reshape(NC, N_ATOM_PAD, MSG), ps[1].reshape(NC, N_ATOM_PAD, MSG),
      ps[2].reshape(NC, N_ATOM_PAD, MSG),
      w1, b1.reshape(1, MSG), w2, b2.reshape(1, MSG), w3, b3.reshape(1, ATOM_DIM))


def kernel(x, bond_x, angle_x, torsion_x, bond_params, angle_params,
           torsion_params, combine_params, bond_src, bond_dst,
           angle_src, angle_dst, torsion_src, torsion_dst):
    params = (bond_params, angle_params, torsion_params)
    w1x = [p[0][1:, :] for p in params]
    v = [p[0][0:1, :] for p in params]
    b1 = [p[1].reshape(1, MSG) for p in params]

    d3 = [jnp.asarray(d, jnp.int32).reshape(NW, CPW, CH)
          for d in (bond_dst, angle_dst, torsion_dst)]
    s3 = [jnp.asarray(s, jnp.int32).reshape(NW, CPW, CH)
          for s in (bond_src, angle_src, torsion_src)]
    fxs = [f.reshape(N_FACTOR) for f in (bond_x, angle_x, torsion_x)]

    hs = _atom_tables(x, w1x, b1)
    g = _sc_gather(hs, d3, s3, fxs)
    hgs, tgs = g[0:3], g[3:6]
    blk_t = 3200

    eye = jnp.eye(PK, dtype=_f32)
    ms = []
    for r in range(3):
        w2d = jnp.kron(eye, params[r][2])
        w3d = jnp.kron(eye, params[r][4])
        b2d = jnp.tile(params[r][3].reshape(1, MSG), (1, PK))
        b3d = jnp.tile(params[r][5].reshape(1, MSG), (1, PK))
        v4 = jnp.tile(v[r], (1, PK))
        ms.append(_edge_mlp(hgs[r].reshape(E // PK, PW),
                            tgs[r].reshape(E // blk_t, blk_t // 128, 128),
                            v4, w2d, b2d, w3d, b3d))

    zrows = jnp.zeros((ROWS_PER_TILE, MSG), _f32)
    ps = _sc_scatter([m.reshape(E, MSG) for m in ms], d3, zrows)
    return _combine(ps, combine_params)


# speedup vs baseline: 1.3553x; 1.3553x over previous
import functools

import jax
import jax.numpy as jnp
from jax import lax
from jax.experimental import pallas as pl
from jax.experimental.pallas import tpu as pltpu
from jax.experimental.pallas import tpu_sc as plsc

N_ATOM = 10000
ATOM_DIM = 128
MSG = 64
E = 320000
N_FACTOR = 160000

NC, NS = 2, 16
NW = NC * NS
CH = 80
EPW = E // NW
CPW = EPW // CH
KD = 5
NG = CPW // KD
N_ATOM_PAD = 10240
ROWS_PER_TILE = N_ATOM_PAD // NS

_f32 = jnp.float32


def _atom_table_body(x_ref, wb, wa, wt, bb, ba, bt, hb, ha, ht):
    xv = x_ref[...]
    hb[...] = jnp.dot(xv, wb[...], preferred_element_type=_f32) + bb[...]
    ha[...] = jnp.dot(xv, wa[...], preferred_element_type=_f32) + ba[...]
    ht[...] = jnp.dot(xv, wt[...], preferred_element_type=_f32) + bt[...]


def _atom_tables(x, ws, bs):
    blk = 1000
    grid = (N_ATOM // blk,)
    wspec = pl.BlockSpec((ATOM_DIM, MSG), lambda i: (0, 0))
    bspec = pl.BlockSpec((1, MSG), lambda i: (0, 0))
    hspec = pl.BlockSpec((blk, MSG), lambda i: (i, 0))
    return pl.pallas_call(
        _atom_table_body,
        grid=grid,
        in_specs=[pl.BlockSpec((blk, ATOM_DIM), lambda i: (i, 0))] + [wspec] * 3 + [bspec] * 3,
        out_specs=[hspec] * 3,
        out_shape=[jax.ShapeDtypeStruct((N_ATOM, MSG), _f32)] * 3,
    )(x, *ws, *bs)


def _sc_gather_body(hb, ha, ht, db, da, dt, sb, sa, st, fb, fa, ft,
                    hgb, hga, hgt, tgb, tga, tgt,
                    dstbuf, srcbuf, rowbuf, tfull, rsem, ssem, wsem):
    cid = lax.axis_index("c")
    sid = lax.axis_index("s")
    wid = cid * NS + sid
    for h_hbm, d3, s3, fx_hbm, hg_hbm, tg3 in (
        (hb, db, sb, fb, hgb, tgb),
        (ha, da, sa, fa, hga, tga),
        (ht, dt, st, ft, hgt, tgt),
    ):
        pltpu.sync_copy(d3.at[wid], dstbuf)
        pltpu.sync_copy(s3.at[wid], srcbuf)

        def group(g, _, h_hbm=h_hbm, fx_hbm=fx_hbm, hg_hbm=hg_hbm):
            descs = []
            for b in range(KD):
                i = g * KD + b
                descs.append(pltpu.async_copy(h_hbm.at[dstbuf.at[i]], rowbuf.at[b], rsem))
                descs.append(pltpu.async_copy(fx_hbm.at[srcbuf.at[i]], tfull.at[i], ssem))
            for d in descs:
                d.wait()
            descs = []
            for b in range(KD):
                e0 = (wid * CPW + g * KD + b) * CH
                descs.append(pltpu.async_copy(rowbuf.at[b], hg_hbm.at[pl.ds(e0, CH)], wsem))
            for d in descs:
                d.wait()
            return 0

        lax.fori_loop(0, NG, group, 0)
        pltpu.sync_copy(tfull, tg3.at[wid])


def _sc_gather(hs, d3s, s3s, fxs):
    mesh = plsc.VectorSubcoreMesh(core_axis_name="c", subcore_axis_name="s")
    out_type = (
        [jax.ShapeDtypeStruct((E, MSG), _f32)] * 3
        + [jax.ShapeDtypeStruct((NW, CPW, CH), _f32)] * 3
    )
    f = pl.kernel(
        _sc_gather_body,
        out_type=out_type,
        mesh=mesh,
        compiler_params=pltpu.CompilerParams(use_tc_tiling_on_sc=False),
        scratch_types=[
            pltpu.VMEM((CPW, CH), jnp.int32),
            pltpu.VMEM((CPW, CH), jnp.int32),
            pltpu.VMEM((KD, CH, MSG), _f32),
            pltpu.VMEM((CPW, CH), _f32),
            pltpu.SemaphoreType.DMA,
            pltpu.SemaphoreType.DMA,
            pltpu.SemaphoreType.DMA,
        ],
    )
    return f(*hs, *d3s, *s3s, *fxs)


PK = 2
PW = PK * MSG


def _edge_mlp_body(hg2_ref, t2_ref, bsel_ref, me_ref, mo_ref, pl_ref, pr_ref,
                   v2_ref, w2d_ref, b2d_ref, w3d_ref, b3d_ref, m2_ref):
    hp = hg2_ref[...]
    t2 = t2_ref[0]
    rowb = jnp.dot(bsel_ref[...], t2, preferred_element_type=_f32)
    t_all = (jnp.dot(rowb * me_ref[...], pl_ref[...], preferred_element_type=_f32)
             + jnp.dot(rowb * mo_ref[...], pr_ref[...], preferred_element_type=_f32))
    a1 = jnp.maximum(hp + t_all * v2_ref[...], 0.0)
    a2 = jnp.maximum(jnp.dot(a1, w2d_ref[...], preferred_element_type=_f32) + b2d_ref[...], 0.0)
    m2_ref[...] = jnp.dot(a2, w3d_ref[...], preferred_element_type=_f32) + b3d_ref[...]


def _edge_mlp(hg2, t2, consts, v2, w2d, b2d, w3d, b3d):
    blk = 3200
    hb = blk // PK
    nr = blk // 128
    grid = (E // blk,)
    bsel, me, mo, plm, prm = consts
    wspec = pl.BlockSpec((PW, PW), lambda i: (0, 0))
    bspec = pl.BlockSpec((1, PW), lambda i: (0, 0))
    return pl.pallas_call(
        _edge_mlp_body,
        grid=grid,
        in_specs=[
            pl.BlockSpec((hb, PW), lambda i: (i, 0)),
            pl.BlockSpec((1, nr, 128), lambda i: (i, 0, 0)),
            pl.BlockSpec((hb, nr), lambda i: (0, 0)),
            pl.BlockSpec((hb, 128), lambda i: (0, 0)),
            pl.BlockSpec((hb, 128), lambda i: (0, 0)),
            pl.BlockSpec((128, 128), lambda i: (0, 0)),
            pl.BlockSpec((128, 128), lambda i: (0, 0)),
            bspec, wspec, bspec, wspec, bspec,
        ],
        out_specs=pl.BlockSpec((hb, PW), lambda i: (i, 0)),
        out_shape=jax.ShapeDtypeStruct((E // PK, PW), _f32),
    )(hg2, t2, bsel, me, mo, plm, prm, v2, w2d, b2d, w3d, b3d)


def _edge_consts():
    blk = 3200
    hb = blk // PK
    nr = blk // 128
    k = jnp.arange(hb)
    c = jnp.arange(128)
    bsel = (k[:, None] // MSG == jnp.arange(nr)[None, :]).astype(_f32)
    me = (2 * (k[:, None] % MSG) == c[None, :]).astype(_f32)
    mo = (2 * (k[:, None] % MSG) + 1 == c[None, :]).astype(_f32)
    plm = jnp.broadcast_to((c < MSG).astype(_f32)[None, :], (128, 128))
    prm = jnp.broadcast_to((c >= MSG).astype(_f32)[None, :], (128, 128))
    return bsel, me, mo, plm, prm


def _sc_scatter_body(mb, ma, mt, db, da, dt, zrows,
                     pb, pa, pt,
                     dstbuf, rowbuf, acc, lsem, asem):
    cid = lax.axis_index("c")
    sid = lax.axis_index("s")
    wid = cid * NS + sid
    arow = sid * ROWS_PER_TILE
    for m_hbm, d3, p_hbm in ((mb, db, pb), (ma, da, pa), (mt, dt, pt)):
        pltpu.sync_copy(d3.at[wid], dstbuf)
        pltpu.sync_copy(zrows, acc.at[pl.ds(arow, ROWS_PER_TILE)])
        plsc.subcore_barrier()

        def group(g, _, m_hbm=m_hbm):
            descs = []
            for b in range(KD):
                e0 = (wid * CPW + g * KD + b) * CH
                descs.append(pltpu.async_copy(m_hbm.at[pl.ds(e0, CH)], rowbuf.at[b], lsem))
            for d in descs:
                d.wait()
            descs = []
            for b in range(KD):
                i = g * KD + b
                descs.append(pltpu.async_copy(rowbuf.at[b], acc.at[dstbuf.at[i]], asem, add=True))
            for d in descs:
                d.wait()
            return 0

        lax.fori_loop(0, NG, group, 0)
        plsc.subcore_barrier()
        pltpu.sync_copy(acc.at[pl.ds(arow, ROWS_PER_TILE)],
                        p_hbm.at[pl.ds(cid * N_ATOM_PAD + arow, ROWS_PER_TILE)])


def _sc_scatter(ms, d3s, zrows):
    mesh = plsc.VectorSubcoreMesh(core_axis_name="c", subcore_axis_name="s")
    out_type = [jax.ShapeDtypeStruct((NC * N_ATOM_PAD, MSG), _f32)] * 3
    f = pl.kernel(
        _sc_scatter_body,
        out_type=out_type,
        mesh=mesh,
        compiler_params=pltpu.CompilerParams(use_tc_tiling_on_sc=False),
        scratch_types=[
            pltpu.VMEM((CPW, CH), jnp.int32),
            pltpu.VMEM((KD, CH, MSG), _f32),
            pltpu.VMEM_SHARED((N_ATOM_PAD, MSG), _f32),
            pltpu.SemaphoreType.DMA,
            pltpu.SemaphoreType.DMA,
        ],
    )
    return f(*ms, *d3s, zrows)


def _combine_body(pb, pa, pt, w1, b1, w2, b2, w3, b3, out):
    ab = pb[0] + pb[1]
    aa = pa[0] + pa[1]
    at = pt[0] + pt[1]
    y = (jnp.dot(ab, w1[0:MSG, :], preferred_element_type=_f32)
         + jnp.dot(aa, w1[MSG:2 * MSG, :], preferred_element_type=_f32)
         + jnp.dot(at, w1[2 * MSG:3 * MSG, :], preferred_element_type=_f32)
         + b1[...])
    y = jnp.maximum(y, 0.0)
    y = jnp.maximum(jnp.dot(y, w2[...], preferred_element_type=_f32) + b2[...], 0.0)
    out[...] = jnp.dot(y, w3[...], preferred_element_type=_f32) + b3[...]


def _combine(ps, cw):
    w1, b1, w2, b2, w3, b3 = cw
    blk = 1000
    grid = (N_ATOM // blk,)
    pspec = pl.BlockSpec((NC, blk, MSG), lambda i: (0, i, 0))
    return pl.pallas_call(
        _combine_body,
        grid=grid,
        in_specs=[pspec, pspec, pspec,
                  pl.BlockSpec((3 * MSG, MSG), lambda i: (0, 0)),
                  pl.BlockSpec((1, MSG), lambda i: (0, 0)),
                  pl.BlockSpec((MSG, MSG), lambda i: (0, 0)),
                  pl.BlockSpec((1, MSG), lambda i: (0, 0)),
                  pl.BlockSpec((MSG, ATOM_DIM), lambda i: (0, 0)),
                  pl.BlockSpec((1, ATOM_DIM), lambda i: (0, 0))],
        out_specs=pl.BlockSpec((blk, ATOM_DIM), lambda i: (i, 0)),
        out_shape=jax.ShapeDtypeStruct((N_ATOM, ATOM_DIM), _f32),
    )(ps[0].reshape(NC, N_ATOM_PAD, MSG), ps[1].reshape(NC, N_ATOM_PAD, MSG),
      ps[2].reshape(NC, N_ATOM_PAD, MSG),
      w1, b1.reshape(1, MSG), w2, b2.reshape(1, MSG), w3, b3.reshape(1, ATOM_DIM))


def kernel(x, bond_x, angle_x, torsion_x, bond_params, angle_params,
           torsion_params, combine_params, bond_src, bond_dst,
           angle_src, angle_dst, torsion_src, torsion_dst):
    params = (bond_params, angle_params, torsion_params)
    w1x = [p[0][1:, :] for p in params]
    v = [p[0][0:1, :] for p in params]
    b1 = [p[1].reshape(1, MSG) for p in params]

    d3 = [jnp.asarray(d, jnp.int32).reshape(NW, CPW, CH)
          for d in (bond_dst, angle_dst, torsion_dst)]
    s3 = [jnp.asarray(s, jnp.int32).reshape(NW, CPW, CH)
          for s in (bond_src, angle_src, torsion_src)]
    fxs = [f.reshape(N_FACTOR) for f in (bond_x, angle_x, torsion_x)]

    hs = _atom_tables(x, w1x, b1)
    g = _sc_gather(hs, d3, s3, fxs)
    hgs, tgs = g[0:3], g[3:6]
    blk_t = 3200

    eye = jnp.eye(PK, dtype=_f32)
    consts = _edge_consts()
    ms = []
    for r in range(3):
        w2d = jnp.kron(eye, params[r][2])
        w3d = jnp.kron(eye, params[r][4])
        b2d = jnp.tile(params[r][3].reshape(1, MSG), (1, PK))
        b3d = jnp.tile(params[r][5].reshape(1, MSG), (1, PK))
        v2 = jnp.tile(v[r], (1, PK))
        ms.append(_edge_mlp(hgs[r].reshape(E // PK, PW),
                            tgs[r].reshape(E // blk_t, blk_t // 128, 128),
                            consts, v2, w2d, b2d, w3d, b3d))

    zrows = jnp.zeros((ROWS_PER_TILE, MSG), _f32)
    ps = _sc_scatter([m.reshape(E, MSG) for m in ms], d3, zrows)
    return _combine(ps, combine_params)


# speedup vs baseline: 1.8653x; 1.3764x over previous
import functools

import jax
import jax.numpy as jnp
from jax import lax
from jax.experimental import pallas as pl
from jax.experimental.pallas import tpu as pltpu
from jax.experimental.pallas import tpu_sc as plsc

N_ATOM = 10000
ATOM_DIM = 128
MSG = 64
E = 320000
N_FACTOR = 160000

NC, NS = 2, 16
NW = NC * NS
CH = 80
EPW = E // NW
CPW = EPW // CH
KD = 5
NG = CPW // KD
N_ATOM_PAD = 10240
ROWS_PER_TILE = N_ATOM_PAD // NS

_f32 = jnp.float32


def _atom_table_body(x_ref, wb, wa, wt, bb, ba, bt, hb, ha, ht):
    xv = x_ref[...]
    hb[...] = jnp.dot(xv, wb[...], preferred_element_type=_f32) + bb[...]
    ha[...] = jnp.dot(xv, wa[...], preferred_element_type=_f32) + ba[...]
    ht[...] = jnp.dot(xv, wt[...], preferred_element_type=_f32) + bt[...]


def _atom_tables(x, ws, bs):
    blk = 1000
    grid = (N_ATOM // blk,)
    wspec = pl.BlockSpec((ATOM_DIM, MSG), lambda i: (0, 0))
    bspec = pl.BlockSpec((1, MSG), lambda i: (0, 0))
    hspec = pl.BlockSpec((blk, MSG), lambda i: (i, 0))
    return pl.pallas_call(
        _atom_table_body,
        grid=grid,
        in_specs=[pl.BlockSpec((blk, ATOM_DIM), lambda i: (i, 0))] + [wspec] * 3 + [bspec] * 3,
        out_specs=[hspec] * 3,
        out_shape=[jax.ShapeDtypeStruct((N_ATOM, MSG), _f32)] * 3,
    )(x, *ws, *bs)


def _sc_gather_body(h_hbm, d3, s3, fx_hbm, hg_hbm, tg3,
                    dstbuf, srcbuf, rowbuf, tfull, rsem, ssem, wsem):
    cid = lax.axis_index("c")
    sid = lax.axis_index("s")
    wid = cid * NS + sid
    pltpu.sync_copy(d3.at[wid], dstbuf)
    pltpu.sync_copy(s3.at[wid], srcbuf)

    def group(g, _):
        descs = []
        for b in range(KD):
            i = g * KD + b
            descs.append(pltpu.async_copy(h_hbm.at[dstbuf.at[i]], rowbuf.at[b], rsem))
            descs.append(pltpu.async_copy(fx_hbm.at[srcbuf.at[i]], tfull.at[i], ssem))
        for d in descs:
            d.wait()
        descs = []
        for b in range(KD):
            e0 = (wid * CPW + g * KD + b) * CH
            descs.append(pltpu.async_copy(rowbuf.at[b], hg_hbm.at[pl.ds(e0, CH)], wsem))
        for d in descs:
            d.wait()
        return 0

    lax.fori_loop(0, NG, group, 0)
    pltpu.sync_copy(tfull, tg3.at[wid])


def _sc_gather(h, d3, s3, fx):
    mesh = plsc.VectorSubcoreMesh(core_axis_name="c", subcore_axis_name="s")
    out_type = (
        jax.ShapeDtypeStruct((E, MSG), _f32),
        jax.ShapeDtypeStruct((NW, CPW, CH), _f32),
    )
    f = pl.kernel(
        _sc_gather_body,
        out_type=out_type,
        mesh=mesh,
        compiler_params=pltpu.CompilerParams(use_tc_tiling_on_sc=False),
        scratch_types=[
            pltpu.VMEM((CPW, CH), jnp.int32),
            pltpu.VMEM((CPW, CH), jnp.int32),
            pltpu.VMEM((KD, CH, MSG), _f32),
            pltpu.VMEM((CPW, CH), _f32),
            pltpu.SemaphoreType.DMA,
            pltpu.SemaphoreType.DMA,
            pltpu.SemaphoreType.DMA,
        ],
    )
    return f(h, d3, s3, fx)


PK = 2
PW = PK * MSG


def _edge_mlp_body(hg2_ref, t2_ref, bsel_ref, me_ref, mo_ref, pl_ref, pr_ref,
                   v2_ref, w2d_ref, b2d_ref, w3d_ref, b3d_ref, m2_ref):
    hp = hg2_ref[...]
    t2 = t2_ref[0]
    rowb = jnp.dot(bsel_ref[...], t2, preferred_element_type=_f32)
    t_all = (jnp.dot(rowb * me_ref[...], pl_ref[...], preferred_element_type=_f32)
             + jnp.dot(rowb * mo_ref[...], pr_ref[...], preferred_element_type=_f32))
    a1 = jnp.maximum(hp + t_all * v2_ref[...], 0.0)
    a2 = jnp.maximum(jnp.dot(a1, w2d_ref[...], preferred_element_type=_f32) + b2d_ref[...], 0.0)
    m2_ref[...] = jnp.dot(a2, w3d_ref[...], preferred_element_type=_f32) + b3d_ref[...]


def _edge_mlp(hg2, t2, consts, v2, w2d, b2d, w3d, b3d):
    blk = 3200
    hb = blk // PK
    nr = blk // 128
    grid = (E // blk,)
    bsel, me, mo, plm, prm = consts
    wspec = pl.BlockSpec((PW, PW), lambda i: (0, 0))
    bspec = pl.BlockSpec((1, PW), lambda i: (0, 0))
    return pl.pallas_call(
        _edge_mlp_body,
        grid=grid,
        in_specs=[
            pl.BlockSpec((hb, PW), lambda i: (i, 0)),
            pl.BlockSpec((1, nr, 128), lambda i: (i, 0, 0)),
            pl.BlockSpec((hb, nr), lambda i: (0, 0)),
            pl.BlockSpec((hb, 128), lambda i: (0, 0)),
            pl.BlockSpec((hb, 128), lambda i: (0, 0)),
            pl.BlockSpec((128, 128), lambda i: (0, 0)),
            pl.BlockSpec((128, 128), lambda i: (0, 0)),
            bspec, wspec, bspec, wspec, bspec,
        ],
        out_specs=pl.BlockSpec((hb, PW), lambda i: (i, 0)),
        out_shape=jax.ShapeDtypeStruct((E // PK, PW), _f32),
    )(hg2, t2, bsel, me, mo, plm, prm, v2, w2d, b2d, w3d, b3d)


def _edge_consts():
    blk = 3200
    hb = blk // PK
    nr = blk // 128
    k = jnp.arange(hb)
    c = jnp.arange(128)
    bsel = (k[:, None] // MSG == jnp.arange(nr)[None, :]).astype(_f32)
    me = (2 * (k[:, None] % MSG) == c[None, :]).astype(_f32)
    mo = (2 * (k[:, None] % MSG) + 1 == c[None, :]).astype(_f32)
    plm = jnp.broadcast_to((c < MSG).astype(_f32)[None, :], (128, 128))
    prm = jnp.broadcast_to((c >= MSG).astype(_f32)[None, :], (128, 128))
    return bsel, me, mo, plm, prm


def _sc_scatter_body(m_hbm, d3, zrows, p_hbm, dstbuf, rowbuf, acc, lsem, asem):
    cid = lax.axis_index("c")
    sid = lax.axis_index("s")
    wid = cid * NS + sid
    arow = sid * ROWS_PER_TILE
    pltpu.sync_copy(d3.at[wid], dstbuf)
    pltpu.sync_copy(zrows, acc.at[pl.ds(arow, ROWS_PER_TILE)])
    plsc.subcore_barrier()

    def group(g, _):
        descs = []
        for b in range(KD):
            e0 = (wid * CPW + g * KD + b) * CH
            descs.append(pltpu.async_copy(m_hbm.at[pl.ds(e0, CH)], rowbuf.at[b], lsem))
        for d in descs:
            d.wait()
        descs = []
        for b in range(KD):
            i = g * KD + b
            descs.append(pltpu.async_copy(rowbuf.at[b], acc.at[dstbuf.at[i]], asem, add=True))
        for d in descs:
            d.wait()
        return 0

    lax.fori_loop(0, NG, group, 0)
    plsc.subcore_barrier()
    pltpu.sync_copy(acc.at[pl.ds(arow, ROWS_PER_TILE)],
                    p_hbm.at[pl.ds(cid * N_ATOM_PAD + arow, ROWS_PER_TILE)])


def _sc_scatter(m, d3, zrows):
    mesh = plsc.VectorSubcoreMesh(core_axis_name="c", subcore_axis_name="s")
    out_type = jax.ShapeDtypeStruct((NC * N_ATOM_PAD, MSG), _f32)
    f = pl.kernel(
        _sc_scatter_body,
        out_type=out_type,
        mesh=mesh,
        compiler_params=pltpu.CompilerParams(use_tc_tiling_on_sc=False),
        scratch_types=[
            pltpu.VMEM((CPW, CH), jnp.int32),
            pltpu.VMEM((KD, CH, MSG), _f32),
            pltpu.VMEM_SHARED((N_ATOM_PAD, MSG), _f32),
            pltpu.SemaphoreType.DMA,
            pltpu.SemaphoreType.DMA,
        ],
    )
    return f(m, d3, zrows)


def _combine_body(pb, pa, pt, w1, b1, w2, b2, w3, b3, out):
    ab = pb[0] + pb[1]
    aa = pa[0] + pa[1]
    at = pt[0] + pt[1]
    y = (jnp.dot(ab, w1[0:MSG, :], preferred_element_type=_f32)
         + jnp.dot(aa, w1[MSG:2 * MSG, :], preferred_element_type=_f32)
         + jnp.dot(at, w1[2 * MSG:3 * MSG, :], preferred_element_type=_f32)
         + b1[...])
    y = jnp.maximum(y, 0.0)
    y = jnp.maximum(jnp.dot(y, w2[...], preferred_element_type=_f32) + b2[...], 0.0)
    out[...] = jnp.dot(y, w3[...], preferred_element_type=_f32) + b3[...]


def _combine(ps, cw):
    w1, b1, w2, b2, w3, b3 = cw
    blk = 1000
    grid = (N_ATOM // blk,)
    pspec = pl.BlockSpec((NC, blk, MSG), lambda i: (0, i, 0))
    return pl.pallas_call(
        _combine_body,
        grid=grid,
        in_specs=[pspec, pspec, pspec,
                  pl.BlockSpec((3 * MSG, MSG), lambda i: (0, 0)),
                  pl.BlockSpec((1, MSG), lambda i: (0, 0)),
                  pl.BlockSpec((MSG, MSG), lambda i: (0, 0)),
                  pl.BlockSpec((1, MSG), lambda i: (0, 0)),
                  pl.BlockSpec((MSG, ATOM_DIM), lambda i: (0, 0)),
                  pl.BlockSpec((1, ATOM_DIM), lambda i: (0, 0))],
        out_specs=pl.BlockSpec((blk, ATOM_DIM), lambda i: (i, 0)),
        out_shape=jax.ShapeDtypeStruct((N_ATOM, ATOM_DIM), _f32),
    )(ps[0].reshape(NC, N_ATOM_PAD, MSG), ps[1].reshape(NC, N_ATOM_PAD, MSG),
      ps[2].reshape(NC, N_ATOM_PAD, MSG),
      w1, b1.reshape(1, MSG), w2, b2.reshape(1, MSG), w3, b3.reshape(1, ATOM_DIM))


def kernel(x, bond_x, angle_x, torsion_x, bond_params, angle_params,
           torsion_params, combine_params, bond_src, bond_dst,
           angle_src, angle_dst, torsion_src, torsion_dst):
    params = (bond_params, angle_params, torsion_params)
    w1x = [p[0][1:, :] for p in params]
    v = [p[0][0:1, :] for p in params]
    b1 = [p[1].reshape(1, MSG) for p in params]

    d3 = [jnp.asarray(d, jnp.int32).reshape(NW, CPW, CH)
          for d in (bond_dst, angle_dst, torsion_dst)]
    s3 = [jnp.asarray(s, jnp.int32).reshape(NW, CPW, CH)
          for s in (bond_src, angle_src, torsion_src)]
    fxs = [f.reshape(N_FACTOR) for f in (bond_x, angle_x, torsion_x)]

    hs = _atom_tables(x, w1x, b1)
    blk_t = 3200
    eye = jnp.eye(PK, dtype=_f32)
    consts = _edge_consts()
    zrows = jnp.zeros((ROWS_PER_TILE, MSG), _f32)
    ps = []
    for r in range(3):
        hg, tg3 = _sc_gather(hs[r], d3[r], s3[r], fxs[r])
        w2d = jnp.kron(eye, params[r][2])
        w3d = jnp.kron(eye, params[r][4])
        b2d = jnp.tile(params[r][3].reshape(1, MSG), (1, PK))
        b3d = jnp.tile(params[r][5].reshape(1, MSG), (1, PK))
        v2 = jnp.tile(v[r], (1, PK))
        m = _edge_mlp(hg.reshape(E // PK, PW),
                      tg3.reshape(E // blk_t, blk_t // 128, 128),
                      consts, v2, w2d, b2d, w3d, b3d)
        ps.append(_sc_scatter(m.reshape(E, MSG), d3[r], zrows))
    return _combine(ps, combine_params)


# speedup vs baseline: 1.8659x; 1.0003x over previous
import functools

import jax
import jax.numpy as jnp
from jax import lax
from jax.experimental import pallas as pl
from jax.experimental.pallas import tpu as pltpu
from jax.experimental.pallas import tpu_sc as plsc

N_ATOM = 10000
ATOM_DIM = 128
MSG = 64
E = 320000
N_FACTOR = 160000

NC, NS = 2, 16
NW = NC * NS
CH = 80
EPW = E // NW
CPW = EPW // CH
KD = 5
NG = CPW // KD
N_ATOM_PAD = 10240
ROWS_PER_TILE = N_ATOM_PAD // NS

_f32 = jnp.float32


def _atom_table_body(x_ref, wb, wa, wt, bb, ba, bt, hb, ha, ht):
    xv = x_ref[...]
    hb[...] = jnp.dot(xv, wb[...], preferred_element_type=_f32) + bb[...]
    ha[...] = jnp.dot(xv, wa[...], preferred_element_type=_f32) + ba[...]
    ht[...] = jnp.dot(xv, wt[...], preferred_element_type=_f32) + bt[...]


def _atom_tables(x, ws, bs):
    blk = 1000
    grid = (N_ATOM // blk,)
    wspec = pl.BlockSpec((ATOM_DIM, MSG), lambda i: (0, 0))
    bspec = pl.BlockSpec((1, MSG), lambda i: (0, 0))
    hspec = pl.BlockSpec((blk, MSG), lambda i: (i, 0))
    return pl.pallas_call(
        _atom_table_body,
        grid=grid,
        in_specs=[pl.BlockSpec((blk, ATOM_DIM), lambda i: (i, 0))] + [wspec] * 3 + [bspec] * 3,
        out_specs=[hspec] * 3,
        out_shape=[jax.ShapeDtypeStruct((N_ATOM, MSG), _f32)] * 3,
    )(x, *ws, *bs)


def _sc_gather_body(h_hbm, d3, s3, fx_hbm, hg_hbm, tg3,
                    dstbuf, srcbuf, rowbuf, tfull, rsem, ssem, wsem):
    cid = lax.axis_index("c")
    sid = lax.axis_index("s")
    wid = cid * NS + sid
    pltpu.sync_copy(d3.at[wid], dstbuf)
    pltpu.sync_copy(s3.at[wid], srcbuf)

    def group(g, _):
        descs = []
        for b in range(KD):
            i = g * KD + b
            descs.append(pltpu.async_copy(h_hbm.at[dstbuf.at[i]], rowbuf.at[b], rsem))
            descs.append(pltpu.async_copy(fx_hbm.at[srcbuf.at[i]], tfull.at[i], ssem))
        for d in descs:
            d.wait()
        descs = []
        for b in range(KD):
            e0 = (wid * CPW + g * KD + b) * CH
            descs.append(pltpu.async_copy(rowbuf.at[b], hg_hbm.at[pl.ds(e0, CH)], wsem))
        for d in descs:
            d.wait()
        return 0

    lax.fori_loop(0, NG, group, 0)
    pltpu.sync_copy(tfull, tg3.at[wid])


def _sc_gather(h, d3, s3, fx):
    mesh = plsc.VectorSubcoreMesh(core_axis_name="c", subcore_axis_name="s")
    out_type = (
        jax.ShapeDtypeStruct((E, MSG), _f32),
        jax.ShapeDtypeStruct((NW, CPW, CH), _f32),
    )
    f = pl.kernel(
        _sc_gather_body,
        out_type=out_type,
        mesh=mesh,
        compiler_params=pltpu.CompilerParams(use_tc_tiling_on_sc=False),
        scratch_types=[
            pltpu.VMEM((CPW, CH), jnp.int32),
            pltpu.VMEM((CPW, CH), jnp.int32),
            pltpu.VMEM((KD, CH, MSG), _f32),
            pltpu.VMEM((CPW, CH), _f32),
            pltpu.SemaphoreType.DMA,
            pltpu.SemaphoreType.DMA,
            pltpu.SemaphoreType.DMA,
        ],
    )
    return f(h, d3, s3, fx)


PK = 2
PW = PK * MSG


def _edge_mlp_body(hg2_ref, t2_ref, bsel_ref, me_ref, mo_ref, pl_ref, pr_ref,
                   v2_ref, w2d_ref, b2d_ref, w3d_ref, b3d_ref, m2_ref):
    hp = hg2_ref[...]
    t2 = t2_ref[0]
    rowb = jnp.dot(bsel_ref[...], t2, preferred_element_type=_f32)
    t_all = (jnp.dot(rowb * me_ref[...], pl_ref[...], preferred_element_type=_f32)
             + jnp.dot(rowb * mo_ref[...], pr_ref[...], preferred_element_type=_f32))
    a1 = jnp.maximum(hp + t_all * v2_ref[...], 0.0).astype(jnp.bfloat16)
    a2 = jnp.maximum(
        jnp.dot(a1, w2d_ref[...].astype(jnp.bfloat16), preferred_element_type=_f32)
        + b2d_ref[...], 0.0).astype(jnp.bfloat16)
    m2_ref[...] = jnp.dot(a2, w3d_ref[...].astype(jnp.bfloat16),
                          preferred_element_type=_f32) + b3d_ref[...]


def _edge_mlp(hg2, t2, consts, v2, w2d, b2d, w3d, b3d):
    blk = 3200
    hb = blk // PK
    nr = blk // 128
    grid = (E // blk,)
    bsel, me, mo, plm, prm = consts
    wspec = pl.BlockSpec((PW, PW), lambda i: (0, 0))
    bspec = pl.BlockSpec((1, PW), lambda i: (0, 0))
    return pl.pallas_call(
        _edge_mlp_body,
        grid=grid,
        in_specs=[
            pl.BlockSpec((hb, PW), lambda i: (i, 0)),
            pl.BlockSpec((1, nr, 128), lambda i: (i, 0, 0)),
            pl.BlockSpec((hb, nr), lambda i: (0, 0)),
            pl.BlockSpec((hb, 128), lambda i: (0, 0)),
            pl.BlockSpec((hb, 128), lambda i: (0, 0)),
            pl.BlockSpec((128, 128), lambda i: (0, 0)),
            pl.BlockSpec((128, 128), lambda i: (0, 0)),
            bspec, wspec, bspec, wspec, bspec,
        ],
        out_specs=pl.BlockSpec((hb, PW), lambda i: (i, 0)),
        out_shape=jax.ShapeDtypeStruct((E // PK, PW), _f32),
    )(hg2, t2, bsel, me, mo, plm, prm, v2, w2d, b2d, w3d, b3d)


def _edge_consts():
    blk = 3200
    hb = blk // PK
    nr = blk // 128
    k = jnp.arange(hb)
    c = jnp.arange(128)
    bsel = (k[:, None] // MSG == jnp.arange(nr)[None, :]).astype(_f32)
    me = (2 * (k[:, None] % MSG) == c[None, :]).astype(_f32)
    mo = (2 * (k[:, None] % MSG) + 1 == c[None, :]).astype(_f32)
    plm = jnp.broadcast_to((c < MSG).astype(_f32)[None, :], (128, 128))
    prm = jnp.broadcast_to((c >= MSG).astype(_f32)[None, :], (128, 128))
    return bsel, me, mo, plm, prm


def _sc_scatter_body(m_hbm, d3, zrows, p_hbm, dstbuf, rowbuf, acc, lsem, asem):
    cid = lax.axis_index("c")
    sid = lax.axis_index("s")
    wid = cid * NS + sid
    arow = sid * ROWS_PER_TILE
    pltpu.sync_copy(d3.at[wid], dstbuf)
    pltpu.sync_copy(zrows, acc.at[pl.ds(arow, ROWS_PER_TILE)])
    plsc.subcore_barrier()

    def group(g, _):
        descs = []
        for b in range(KD):
            e0 = (wid * CPW + g * KD + b) * CH
            descs.append(pltpu.async_copy(m_hbm.at[pl.ds(e0, CH)], rowbuf.at[b], lsem))
        for d in descs:
            d.wait()
        descs = []
        for b in range(KD):
            i = g * KD + b
            descs.append(pltpu.async_copy(rowbuf.at[b], acc.at[dstbuf.at[i]], asem, add=True))
        for d in descs:
            d.wait()
        return 0

    lax.fori_loop(0, NG, group, 0)
    plsc.subcore_barrier()
    pltpu.sync_copy(acc.at[pl.ds(arow, ROWS_PER_TILE)],
                    p_hbm.at[pl.ds(cid * N_ATOM_PAD + arow, ROWS_PER_TILE)])


def _sc_scatter(m, d3, zrows):
    mesh = plsc.VectorSubcoreMesh(core_axis_name="c", subcore_axis_name="s")
    out_type = jax.ShapeDtypeStruct((NC * N_ATOM_PAD, MSG), _f32)
    f = pl.kernel(
        _sc_scatter_body,
        out_type=out_type,
        mesh=mesh,
        compiler_params=pltpu.CompilerParams(use_tc_tiling_on_sc=False),
        scratch_types=[
            pltpu.VMEM((CPW, CH), jnp.int32),
            pltpu.VMEM((KD, CH, MSG), _f32),
            pltpu.VMEM_SHARED((N_ATOM_PAD, MSG), _f32),
            pltpu.SemaphoreType.DMA,
            pltpu.SemaphoreType.DMA,
        ],
    )
    return f(m, d3, zrows)


def _combine_body(pb, pa, pt, w1, b1, w2, b2, w3, b3, out):
    ab = pb[0] + pb[1]
    aa = pa[0] + pa[1]
    at = pt[0] + pt[1]
    y = (jnp.dot(ab, w1[0:MSG, :], preferred_element_type=_f32)
         + jnp.dot(aa, w1[MSG:2 * MSG, :], preferred_element_type=_f32)
         + jnp.dot(at, w1[2 * MSG:3 * MSG, :], preferred_element_type=_f32)
         + b1[...])
    y = jnp.maximum(y, 0.0)
    y = jnp.maximum(jnp.dot(y, w2[...], preferred_element_type=_f32) + b2[...], 0.0)
    out[...] = jnp.dot(y, w3[...], preferred_element_type=_f32) + b3[...]


def _combine(ps, cw):
    w1, b1, w2, b2, w3, b3 = cw
    blk = 1000
    grid = (N_ATOM // blk,)
    pspec = pl.BlockSpec((NC, blk, MSG), lambda i: (0, i, 0))
    return pl.pallas_call(
        _combine_body,
        grid=grid,
        in_specs=[pspec, pspec, pspec,
                  pl.BlockSpec((3 * MSG, MSG), lambda i: (0, 0)),
                  pl.BlockSpec((1, MSG), lambda i: (0, 0)),
                  pl.BlockSpec((MSG, MSG), lambda i: (0, 0)),
                  pl.BlockSpec((1, MSG), lambda i: (0, 0)),
                  pl.BlockSpec((MSG, ATOM_DIM), lambda i: (0, 0)),
                  pl.BlockSpec((1, ATOM_DIM), lambda i: (0, 0))],
        out_specs=pl.BlockSpec((blk, ATOM_DIM), lambda i: (i, 0)),
        out_shape=jax.ShapeDtypeStruct((N_ATOM, ATOM_DIM), _f32),
    )(ps[0].reshape(NC, N_ATOM_PAD, MSG), ps[1].reshape(NC, N_ATOM_PAD, MSG),
      ps[2].reshape(NC, N_ATOM_PAD, MSG),
      w1, b1.reshape(1, MSG), w2, b2.reshape(1, MSG), w3, b3.reshape(1, ATOM_DIM))


def kernel(x, bond_x, angle_x, torsion_x, bond_params, angle_params,
           torsion_params, combine_params, bond_src, bond_dst,
           angle_src, angle_dst, torsion_src, torsion_dst):
    params = (bond_params, angle_params, torsion_params)
    w1x = [p[0][1:, :] for p in params]
    v = [p[0][0:1, :] for p in params]
    b1 = [p[1].reshape(1, MSG) for p in params]

    d3 = [jnp.asarray(d, jnp.int32).reshape(NW, CPW, CH)
          for d in (bond_dst, angle_dst, torsion_dst)]
    s3 = [jnp.asarray(s, jnp.int32).reshape(NW, CPW, CH)
          for s in (bond_src, angle_src, torsion_src)]
    fxs = [f.reshape(N_FACTOR) for f in (bond_x, angle_x, torsion_x)]

    hs = _atom_tables(x, w1x, b1)
    blk_t = 3200
    eye = jnp.eye(PK, dtype=_f32)
    consts = _edge_consts()
    zrows = jnp.zeros((ROWS_PER_TILE, MSG), _f32)
    ps = []
    for r in range(3):
        hg, tg3 = _sc_gather(hs[r], d3[r], s3[r], fxs[r])
        w2d = jnp.kron(eye, params[r][2])
        w3d = jnp.kron(eye, params[r][4])
        b2d = jnp.tile(params[r][3].reshape(1, MSG), (1, PK))
        b3d = jnp.tile(params[r][5].reshape(1, MSG), (1, PK))
        v2 = jnp.tile(v[r], (1, PK))
        m = _edge_mlp(hg.reshape(E // PK, PW),
                      tg3.reshape(E // blk_t, blk_t // 128, 128),
                      consts, v2, w2d, b2d, w3d, b3d)
        ps.append(_sc_scatter(m.reshape(E, MSG), d3[r], zrows))
    return _combine(ps, combine_params)
